# baseline (speedup 1.0000x reference)
import functools

import jax
import jax.numpy as jnp
from jax import lax
from jax.experimental import pallas as pl
from jax.experimental.pallas import tpu as pltpu
from jax.experimental.pallas import tpu_sc as plsc

_PERC = 0.25
_L = 16
_NSUB = 32
_NBIN = 256


def _ukey(x, int_min):
    bits = plsc.bitcast(x, jnp.int32)
    key = jnp.where(bits >= 0, bits, int_min - bits)
    return key ^ int_min


def _srl(v, n):
    return lax.shift_right_logical(v, jnp.full((_L,), n, jnp.int32))


def _zero_hists(hcnt, hsum):
    zi = jnp.zeros((_L,), jnp.int32)
    zf = jnp.zeros((_L,), jnp.float32)

    def body(i, c):
        hcnt[pl.ds(i * _L, _L)] = zi
        hsum[pl.ds(i * _L, _L)] = zf
        return c

    lax.fori_loop(0, _NBIN * _L // _L, body, 0)


def _level_scan(hcnt, hsum, k_cur, lane_iota):
    best_bin = jnp.int32(-1)
    best_A = jnp.int32(0)
    best_As = jnp.float32(0.0)
    carry = jnp.int32(0)
    carry_s = jnp.float32(0.0)
    for g in reversed(range(_NBIN // _L)):

        def lsum(l, acc):
            tc, ts = acc
            off = l * _NBIN + g * _L
            return (tc + hcnt[pl.ds(off, _L)], ts + hsum[pl.ds(off, _L)])

        tot, tots = lax.fori_loop(
            0, _L, lsum, (jnp.zeros((_L,), jnp.int32), jnp.zeros((_L,), jnp.float32))
        )
        S = plsc.cumsum(tot)
        Ss = plsc.cumsum(tots)
        Tg = S[_L - 1]
        Tgs = Ss[_L - 1]
        A = carry + Tg - S
        As = carry_s + Tgs - Ss
        mask = (A < k_cur) & (A + tot >= k_cur)
        ids = g * _L + lane_iota
        best_bin = jnp.maximum(best_bin, jnp.max(jnp.where(mask, ids, -1)))
        best_A = jnp.maximum(best_A, jnp.max(jnp.where(mask, A, -1)))
        best_As = jnp.maximum(best_As, jnp.max(jnp.where(mask, As, -jnp.inf)))
        carry = carry + Tg
        carry_s = carry_s + Tgs
    return best_bin, best_A, best_As


def _sc_body(nrows_per_sub, nchunks, k, loss_hbm, out_hbm, data, compact, hcnt,
             hsum, accv):
    int_min = jnp.int32(-(2**31))
    lane_iota = lax.iota(jnp.int32, _L)
    lane_base = lane_iota * _NBIN
    ones_i = jnp.ones((_L,), jnp.int32)
    wid = lax.axis_index("s") * 2 + lax.axis_index("c")

    def row_body(r, acc):
        row = wid * nrows_per_sub + r
        pltpu.sync_copy(loss_hbm.at[row], data)

        _zero_hists(hcnt, hsum)

        def p1(c, carry):
            x = data[pl.ds(c * _L, _L)]
            u = _ukey(x, int_min)
            idx = lane_base + _srl(u, 24)
            plsc.addupdate_scatter(hcnt, [idx], ones_i)
            plsc.addupdate_scatter(hsum, [idx], x)
            return carry

        lax.fori_loop(0, nchunks, p1, 0)
        b1, A1, As1 = _level_scan(hcnt, hsum, k, lane_iota)
        k2 = k - A1

        _zero_hists(hcnt, hsum)

        def p2(c, n2):
            x = data[pl.ds(c * _L, _L)]
            u = _ukey(x, int_min)
            m = _srl(u, 24) == b1
            idx = lane_base + (_srl(u, 16) & 0xFF)
            plsc.addupdate_scatter(hcnt, [idx], ones_i, mask=m)
            plsc.addupdate_scatter(hsum, [idx], x, mask=m)
            S = plsc.cumsum(m.astype(jnp.int32))
            plsc.store_scatter(compact, [n2 + S - 1], x, mask=m)
            return n2 + S[_L - 1]

        n2 = lax.fori_loop(0, nchunks, p2, jnp.int32(0))
        b2, A2, As2 = _level_scan(hcnt, hsum, k2, lane_iota)
        k3 = k2 - A2

        nch3 = (n2 + (_L - 1)) // _L

        _zero_hists(hcnt, hsum)

        def p3(c, carry):
            base = c * _L
            x = compact[pl.ds(base, _L)]
            valid = lane_iota < (n2 - base)
            u = _ukey(x, int_min)
            m = valid & ((_srl(u, 16) & 0xFF) == b2)
            idx = lane_base + (_srl(u, 8) & 0xFF)
            plsc.addupdate_scatter(hcnt, [idx], ones_i, mask=m)
            plsc.addupdate_scatter(hsum, [idx], x, mask=m)
            return carry

        lax.fori_loop(0, nch3, p3, 0)
        b3, A3, As3 = _level_scan(hcnt, hsum, k3, lane_iota)
        k4 = k3 - A3

        _zero_hists(hcnt, hsum)

        def p4(c, carry):
            base = c * _L
            x = compact[pl.ds(base, _L)]
            valid = lane_iota < (n2 - base)
            u = _ukey(x, int_min)
            m = valid & ((_srl(u, 16) & 0xFF) == b2) & ((_srl(u, 8) & 0xFF) == b3)
            idx = lane_base + (u & 0xFF)
            plsc.addupdate_scatter(hcnt, [idx], ones_i, mask=m)
            plsc.addupdate_scatter(hsum, [idx], x, mask=m)
            return carry

        lax.fori_loop(0, nch3, p4, 0)
        b4, A4, As4 = _level_scan(hcnt, hsum, k4, lane_iota)
        k5 = k4 - A4

        t_u = (b1 << 24) | (b2 << 16) | (b3 << 8) | b4
        t_key = t_u ^ int_min
        t_bits = jnp.where(t_key >= 0, t_key, int_min - t_key)
        t_vec = plsc.bitcast(jnp.full((_L,), t_bits, jnp.int32), jnp.float32)
        t_f = t_vec[0]
        row_sum = As1 + As2 + As3 + As4 + k5.astype(jnp.float32) * t_f
        return acc + row_sum

    acc = lax.fori_loop(0, nrows_per_sub, row_body, jnp.float32(0.0))
    accv[...] = jnp.where(lane_iota == 0, acc, 0.0)
    pltpu.sync_copy(accv, out_hbm.at[wid])


def kernel(loss):
    B = loss.shape[0]
    loss2 = loss.reshape(B, -1)
    P = loss2.shape[1]
    k = int(_PERC * P)
    nrows_per_sub = B // _NSUB
    nchunks = P // _L

    mesh = plsc.VectorSubcoreMesh(core_axis_name="c", subcore_axis_name="s")
    sc_call = pl.kernel(
        functools.partial(_sc_body, nrows_per_sub, nchunks, jnp.int32(k)),
        out_type=jax.ShapeDtypeStruct((_NSUB, _L), jnp.float32),
        mesh=mesh,
        compiler_params=pltpu.CompilerParams(needs_layout_passes=False),
        scratch_types=[
            pltpu.VMEM((P,), jnp.float32),
            pltpu.VMEM((P + _L,), jnp.float32),
            pltpu.VMEM((_NBIN * _L,), jnp.int32),
            pltpu.VMEM((_NBIN * _L,), jnp.float32),
            pltpu.VMEM((_L,), jnp.float32),
        ],
    )
    partial_sums = sc_call(loss2)
    return jnp.sum(partial_sums) / (B * k)


# speedup vs baseline: 1.0982x; 1.0982x over previous
import functools

import jax
import jax.numpy as jnp
from jax import lax
from jax.experimental import pallas as pl
from jax.experimental.pallas import tpu as pltpu
from jax.experimental.pallas import tpu_sc as plsc

_PERC = 0.25
_L = 16
_NSUB = 32
_NBIN = 256
_UNROLL = 8


def _keys(x, int_min):
    bits = plsc.bitcast(x, jnp.int32)
    key = jnp.where(bits >= 0, bits, int_min - bits)
    return key, key ^ int_min


def _srl(v, n):
    return lax.shift_right_logical(v, jnp.full((_L,), n, jnp.int32))


def _zero_hist(hcnt):
    zi = jnp.zeros((_L,), jnp.int32)

    def body(i, c):
        for j in range(_UNROLL):
            hcnt[pl.ds((i * _UNROLL + j) * _L, _L)] = zi
        return c

    lax.fori_loop(0, _NBIN // _UNROLL, body, 0)


def _hist_pass(data, hcnt, nchunks, lane_base, ones_i, int_min, shift,
               prefix_shift=None, prefix=None):
    def body(c, carry):
        for j in range(_UNROLL):
            x = data[pl.ds((c * _UNROLL + j) * _L, _L)]
            _, u = _keys(x, int_min)
            b = _srl(u, shift)
            if shift != 24:
                b = b & 0xFF
            idx = lane_base + b
            if prefix_shift is None:
                plsc.addupdate_scatter(hcnt, [idx], ones_i)
            else:
                m = _srl(u, prefix_shift) == prefix
                plsc.addupdate_scatter(hcnt, [idx], ones_i, mask=m)
        return carry

    lax.fori_loop(0, nchunks // _UNROLL, body, 0)


def _level_scan(hcnt, k_cur, lane_iota):
    best_bin = jnp.int32(-1)
    best_A = jnp.int32(0)
    carry = jnp.int32(0)
    for g in reversed(range(_NBIN // _L)):
        tot = jnp.zeros((_L,), jnp.int32)
        for l in range(_L):
            tot = tot + hcnt[pl.ds(l * _NBIN + g * _L, _L)]
        S = plsc.cumsum(tot)
        Tg = S[_L - 1]
        A = carry + Tg - S
        mask = (A < k_cur) & (A + tot >= k_cur)
        ids = g * _L + lane_iota
        best_bin = jnp.maximum(best_bin, jnp.max(jnp.where(mask, ids, -1)))
        best_A = jnp.maximum(best_A, jnp.max(jnp.where(mask, A, -1)))
        carry = carry + Tg
    return best_bin, best_A


def _sc_body(nrows_per_sub, nchunks, k, loss_hbm, out_hbm, data, hcnt, accv):
    int_min = jnp.int32(-(2**31))
    lane_iota = lax.iota(jnp.int32, _L)
    lane_base = lane_iota * _NBIN
    ones_i = jnp.ones((_L,), jnp.int32)
    wid = lax.axis_index("s") * 2 + lax.axis_index("c")

    def row_body(r, acc):
        row = wid * nrows_per_sub + r
        pltpu.sync_copy(loss_hbm.at[row], data)

        _zero_hist(hcnt)
        _hist_pass(data, hcnt, nchunks, lane_base, ones_i, int_min, 24)
        b1, A1 = _level_scan(hcnt, k, lane_iota)
        k2 = k - A1

        _zero_hist(hcnt)
        _hist_pass(data, hcnt, nchunks, lane_base, ones_i, int_min, 16,
                   prefix_shift=24, prefix=b1)
        b2, A2 = _level_scan(hcnt, k2, lane_iota)
        k3 = k2 - A2
        p16 = (b1 << 8) | b2

        _zero_hist(hcnt)
        _hist_pass(data, hcnt, nchunks, lane_base, ones_i, int_min, 8,
                   prefix_shift=16, prefix=p16)
        b3, A3 = _level_scan(hcnt, k3, lane_iota)
        k4 = k3 - A3
        p24 = (p16 << 8) | b3

        _zero_hist(hcnt)
        _hist_pass(data, hcnt, nchunks, lane_base, ones_i, int_min, 0,
                   prefix_shift=8, prefix=p24)
        b4, A4 = _level_scan(hcnt, k4, lane_iota)
        k5 = k4 - A4

        t_u = (p24 << 8) | b4
        t_key = t_u ^ int_min

        def p5(c, carry):
            sacc, cacc = carry
            for j in range(_UNROLL):
                x = data[pl.ds((c * _UNROLL + j) * _L, _L)]
                key, _ = _keys(x, int_min)
                m = key > t_key
                sacc = sacc + jnp.where(m, x, 0.0)
                cacc = cacc + m.astype(jnp.int32)
            return sacc, cacc

        sacc, cacc = lax.fori_loop(
            0, nchunks // _UNROLL, p5,
            (jnp.zeros((_L,), jnp.float32), jnp.zeros((_L,), jnp.int32)),
        )
        sum_gt = jnp.sum(sacc)
        cnt_gt = jnp.sum(cacc)

        t_bits = jnp.where(t_key >= 0, t_key, int_min - t_key)
        t_vec = plsc.bitcast(jnp.full((_L,), t_bits, jnp.int32), jnp.float32)
        t_f = t_vec[0]
        row_sum = sum_gt + (k - cnt_gt).astype(jnp.float32) * t_f
        return acc + row_sum

    acc = lax.fori_loop(0, nrows_per_sub, row_body, jnp.float32(0.0))
    accv[...] = jnp.where(lane_iota == 0, acc, 0.0)
    pltpu.sync_copy(accv, out_hbm.at[wid])


def kernel(loss):
    B = loss.shape[0]
    loss2 = loss.reshape(B, -1)
    P = loss2.shape[1]
    k = int(_PERC * P)
    nrows_per_sub = B // _NSUB
    nchunks = P // _L

    mesh = plsc.VectorSubcoreMesh(core_axis_name="c", subcore_axis_name="s")
    sc_call = pl.kernel(
        functools.partial(_sc_body, nrows_per_sub, nchunks, jnp.int32(k)),
        out_type=jax.ShapeDtypeStruct((_NSUB, _L), jnp.float32),
        mesh=mesh,
        compiler_params=pltpu.CompilerParams(needs_layout_passes=False),
        scratch_types=[
            pltpu.VMEM((P,), jnp.float32),
            pltpu.VMEM((_NBIN * _L,), jnp.int32),
            pltpu.VMEM((_L,), jnp.float32),
        ],
    )
    partial_sums = sc_call(loss2)
    return jnp.sum(partial_sums) / (B * k)


# speedup vs baseline: 2.6224x; 2.3879x over previous
import functools

import jax
import jax.numpy as jnp
from jax import lax
from jax.experimental import pallas as pl
from jax.experimental.pallas import tpu as pltpu
from jax.experimental.pallas import tpu_sc as plsc

_PERC = 0.25
_L = 16
_NSUB = 32
_NBIN = 256
_UNROLL = 8


def _keys(x, int_min):
    bits = plsc.bitcast(x, jnp.int32)
    key = jnp.where(bits >= 0, bits, int_min - bits)
    return key, key ^ int_min


def _srl(v, n):
    return lax.shift_right_logical(v, jnp.full((_L,), n, jnp.int32))


def _zero_hist(hcnt):
    zi = jnp.zeros((_L,), jnp.int32)

    @plsc.parallel_loop(0, _NBIN, unroll=_UNROLL)
    def _(i):
        hcnt[pl.ds(i * _L, _L)] = zi


def _hist_pass(data, hcnt, nchunks, lane_base, ones_i, int_min, shift,
               prefix_shift=None, prefix=None):
    @plsc.parallel_loop(0, nchunks, unroll=_UNROLL)
    def _(c):
        x = data[pl.ds(c * _L, _L)]
        _, u = _keys(x, int_min)
        b = _srl(u, shift)
        if shift != 24:
            b = b & 0xFF
        idx = lane_base + b
        if prefix_shift is None:
            plsc.addupdate_scatter(hcnt, [idx], ones_i)
        else:
            m = _srl(u, prefix_shift) == prefix
            plsc.addupdate_scatter(hcnt, [idx], ones_i, mask=m)


def _level_scan(hcnt, k_cur, lane_iota):
    best_bin = jnp.int32(-1)
    best_A = jnp.int32(0)
    carry = jnp.int32(0)
    for g in reversed(range(_NBIN // _L)):
        tot = jnp.zeros((_L,), jnp.int32)
        for l in range(_L):
            tot = tot + hcnt[pl.ds(l * _NBIN + g * _L, _L)]
        S = plsc.cumsum(tot)
        Tg = S[_L - 1]
        A = carry + Tg - S
        mask = (A < k_cur) & (A + tot >= k_cur)
        ids = g * _L + lane_iota
        best_bin = jnp.maximum(best_bin, jnp.max(jnp.where(mask, ids, -1)))
        best_A = jnp.maximum(best_A, jnp.max(jnp.where(mask, A, -1)))
        carry = carry + Tg
    return best_bin, best_A


def _sc_body(nrows_per_sub, nchunks, k, loss_hbm, out_hbm, data, hcnt, accv):
    int_min = jnp.int32(-(2**31))
    lane_iota = lax.iota(jnp.int32, _L)
    lane_base = lane_iota * _NBIN
    ones_i = jnp.ones((_L,), jnp.int32)
    wid = lax.axis_index("s") * 2 + lax.axis_index("c")

    def row_body(r, acc):
        row = wid * nrows_per_sub + r
        pltpu.sync_copy(loss_hbm.at[row], data)

        _zero_hist(hcnt)
        _hist_pass(data, hcnt, nchunks, lane_base, ones_i, int_min, 24)
        b1, A1 = _level_scan(hcnt, k, lane_iota)
        k2 = k - A1

        _zero_hist(hcnt)
        _hist_pass(data, hcnt, nchunks, lane_base, ones_i, int_min, 16,
                   prefix_shift=24, prefix=b1)
        b2, A2 = _level_scan(hcnt, k2, lane_iota)
        k3 = k2 - A2
        p16 = (b1 << 8) | b2

        _zero_hist(hcnt)
        _hist_pass(data, hcnt, nchunks, lane_base, ones_i, int_min, 8,
                   prefix_shift=16, prefix=p16)
        b3, A3 = _level_scan(hcnt, k3, lane_iota)
        k4 = k3 - A3
        p24 = (p16 << 8) | b3

        _zero_hist(hcnt)
        _hist_pass(data, hcnt, nchunks, lane_base, ones_i, int_min, 0,
                   prefix_shift=8, prefix=p24)
        b4, A4 = _level_scan(hcnt, k4, lane_iota)
        k5 = k4 - A4

        t_u = (p24 << 8) | b4
        t_key = t_u ^ int_min

        zero_carry = (jnp.zeros((_L,), jnp.float32), jnp.zeros((_L,), jnp.int32))

        @plsc.parallel_loop(0, nchunks, unroll=_UNROLL, carry=zero_carry)
        def p5_acc(c, carry):
            sacc, cacc = carry
            x = data[pl.ds(c * _L, _L)]
            key, _ = _keys(x, int_min)
            m = key > t_key
            return sacc + jnp.where(m, x, 0.0), cacc + m.astype(jnp.int32)

        sacc, cacc = p5_acc
        sum_gt = jnp.sum(sacc)
        cnt_gt = jnp.sum(cacc)

        t_bits = jnp.where(t_key >= 0, t_key, int_min - t_key)
        t_vec = plsc.bitcast(jnp.full((_L,), t_bits, jnp.int32), jnp.float32)
        t_f = t_vec[0]
        row_sum = sum_gt + (k - cnt_gt).astype(jnp.float32) * t_f
        return acc + row_sum

    acc = lax.fori_loop(0, nrows_per_sub, row_body, jnp.float32(0.0))
    accv[...] = jnp.where(lane_iota == 0, acc, 0.0)
    pltpu.sync_copy(accv, out_hbm.at[wid])


def kernel(loss):
    B = loss.shape[0]
    loss2 = loss.reshape(B, -1)
    P = loss2.shape[1]
    k = int(_PERC * P)
    nrows_per_sub = B // _NSUB
    nchunks = P // _L

    mesh = plsc.VectorSubcoreMesh(core_axis_name="c", subcore_axis_name="s")
    sc_call = pl.kernel(
        functools.partial(_sc_body, nrows_per_sub, nchunks, jnp.int32(k)),
        out_type=jax.ShapeDtypeStruct((_NSUB, _L), jnp.float32),
        mesh=mesh,
        compiler_params=pltpu.CompilerParams(needs_layout_passes=False),
        scratch_types=[
            pltpu.VMEM((P,), jnp.float32),
            pltpu.VMEM((_NBIN * _L,), jnp.int32),
            pltpu.VMEM((_L,), jnp.float32),
        ],
    )
    partial_sums = sc_call(loss2)
    return jnp.sum(partial_sums) / (B * k)
